# Initial kernel scaffold; baseline (speedup 1.0000x reference)
import jax
import jax.numpy as jnp
from jax.experimental import pallas as pl


def kernel(inputs, token_table, position_table):
    raise NotImplementedError("write your pallas kernel here")


# speedup vs baseline: 3.0675x; 3.0675x over previous
import functools

import jax
import jax.numpy as jnp
from jax import lax
from jax.experimental import pallas as pl
from jax.experimental.pallas import tpu as pltpu
from jax.experimental.pallas import tpu_sc as plsc

_NC = 2
_NS = 16
_NW = _NC * _NS
_LANES = 16


@functools.cache
def _make_kernel(batch, seq, emb, n_chunks, chunk):
    rows_per_w = batch // _NW
    mesh = plsc.VectorSubcoreMesh(core_axis_name="c", subcore_axis_name="s")

    @functools.partial(
        pl.kernel,
        out_type=jax.ShapeDtypeStruct((batch, seq, emb), jnp.float32),
        mesh=mesh,
        compiler_params=pltpu.CompilerParams(use_tc_tiling_on_sc=False),
        scratch_types=[
            pltpu.VMEM((seq, emb), jnp.float32),
            pltpu.VMEM((n_chunks, chunk), jnp.int32),
            pltpu.VMEM((seq, emb), jnp.float32),
            pltpu.SemaphoreType.DMA,
        ],
    )
    def emb_kernel(idx_hbm, tok_hbm, pos_hbm, out_hbm, pos_v, idx_v, rows_v, sem):
        wid = lax.axis_index("s") * _NC + lax.axis_index("c")
        pltpu.sync_copy(pos_hbm, pos_v)

        def row_body(r, carry):
            b = wid * rows_per_w + r
            pltpu.sync_copy(idx_hbm.at[b], idx_v)
            cps = [
                pltpu.async_copy(
                    tok_hbm.at[idx_v.at[j]],
                    rows_v.at[pl.ds(j * chunk, chunk)],
                    sem,
                )
                for j in range(n_chunks)
            ]
            for cp in cps:
                cp.wait()

            def add_body(i, c):
                for k in range(emb // _LANES):
                    sl = pl.ds(k * _LANES, _LANES)
                    rows_v[i, sl] = rows_v[i, sl] + pos_v[i, sl]
                return c

            lax.fori_loop(0, seq, add_body, 0)
            pltpu.sync_copy(rows_v, out_hbm.at[b])
            return carry

        lax.fori_loop(0, rows_per_w, row_body, 0)

    return emb_kernel


def kernel(inputs, token_table, position_table):
    batch, seq = inputs.shape
    emb = token_table.shape[1]
    chunk = 100
    n_chunks = seq // chunk
    idx = inputs.astype(jnp.int32).reshape(batch, n_chunks, chunk)
    f = _make_kernel(batch, seq, emb, n_chunks, chunk)
    return f(idx, token_table, position_table)


# speedup vs baseline: 3.4335x; 1.1193x over previous
import functools

import jax
import jax.numpy as jnp
from jax import lax
from jax.experimental import pallas as pl
from jax.experimental.pallas import tpu as pltpu
from jax.experimental.pallas import tpu_sc as plsc

_NC = 2
_NS = 16
_NW = _NC * _NS
_LANES = 16


@functools.cache
def _make_kernel(batch, seq, emb, n_chunks, chunk):
    rows_per_w = batch // _NW
    assert rows_per_w % 2 == 0
    mesh = plsc.VectorSubcoreMesh(core_axis_name="c", subcore_axis_name="s")

    @functools.partial(
        pl.kernel,
        out_type=jax.ShapeDtypeStruct((batch, seq, emb), jnp.float32),
        mesh=mesh,
        compiler_params=pltpu.CompilerParams(use_tc_tiling_on_sc=False),
        scratch_types=[
            pltpu.VMEM((seq, emb), jnp.float32),
            pltpu.VMEM((rows_per_w, n_chunks, chunk), jnp.int32),
            pltpu.VMEM((seq, emb), jnp.float32),
            pltpu.VMEM((seq, emb), jnp.float32),
            pltpu.SemaphoreType.DMA,
            pltpu.SemaphoreType.DMA,
            pltpu.SemaphoreType.DMA,
            pltpu.SemaphoreType.DMA,
        ],
    )
    def emb_kernel(idx_hbm, tok_hbm, pos_hbm, out_hbm,
                   pos_v, idx_all, rows0, rows1, in0, in1, out0, out1):
        wid = lax.axis_index("s") * _NC + lax.axis_index("c")
        base = wid * rows_per_w
        pltpu.sync_copy(idx_hbm.at[pl.ds(base, rows_per_w)], idx_all)
        pltpu.sync_copy(pos_hbm, pos_v)

        rows = (rows0, rows1)
        ins = (in0, in1)
        outs = (out0, out1)

        def gather_cps(r_local, buf):
            return [
                (tok_hbm.at[idx_all.at[r_local, j]],
                 rows[buf].at[pl.ds(j * chunk, chunk)],
                 ins[buf])
                for j in range(n_chunks)
            ]

        def start_gather(r_local, buf):
            for args in gather_cps(r_local, buf):
                pltpu.async_copy(*args)

        def wait_gather(r_local, buf):
            for args in gather_cps(r_local, buf):
                pltpu.make_async_copy(*args).wait()

        def add_pos(buf):
            rv = rows[buf]

            def body(i, c):
                for k in range(emb // _LANES):
                    sl = pl.ds(k * _LANES, _LANES)
                    plsc.addupdate(rv.at[i, sl], pos_v[i, sl])
                return c

            lax.fori_loop(0, seq, body, 0)

        def start_out(r_local, buf):
            pltpu.async_copy(rows[buf], out_hbm.at[base + r_local], outs[buf])

        def wait_out(r_local, buf):
            pltpu.make_async_copy(
                rows[buf], out_hbm.at[base + r_local], outs[buf]).wait()

        start_gather(0, 0)
        start_gather(1, 1)
        wait_gather(0, 0)
        add_pos(0)
        start_out(0, 0)

        def pair(k, c):
            r = 2 * k + 1
            wait_gather(r, 1)
            add_pos(1)
            start_out(r, 1)
            wait_out(r - 1, 0)
            start_gather(r + 1, 0)

            wait_gather(r + 1, 0)
            add_pos(0)
            start_out(r + 1, 0)
            wait_out(r, 1)
            start_gather(r + 2, 1)
            return c

        lax.fori_loop(0, (rows_per_w - 2) // 2, pair, 0)

        rl = rows_per_w - 1
        wait_gather(rl, 1)
        add_pos(1)
        start_out(rl, 1)
        wait_out(rl - 1, 0)
        wait_out(rl, 1)

    return emb_kernel


def kernel(inputs, token_table, position_table):
    batch, seq = inputs.shape
    emb = token_table.shape[1]
    chunk = 100
    n_chunks = seq // chunk
    idx = inputs.astype(jnp.int32).reshape(batch, n_chunks, chunk)
    f = _make_kernel(batch, seq, emb, n_chunks, chunk)
    return f(idx, token_table, position_table)
